# baseline (speedup 1.0000x reference)
import functools

import jax
import jax.numpy as jnp
from jax import lax
from jax.experimental import pallas as pl
from jax.experimental.pallas import tpu as pltpu
from jax.experimental.pallas import tpu_sc as plsc

_B, _N, _D = 1024, 1000, 128


def _sc_gather(current_node, psi_flat, coords_flat):
    info = plsc.get_sparse_core_info()
    nc, ns, nl = info.num_cores, info.num_subcores, info.num_lanes
    nw = nc * ns
    bpw = _B // nw

    mesh = plsc.VectorSubcoreMesh(core_axis_name="c", subcore_axis_name="s")

    @functools.partial(
        pl.kernel,
        out_type=(
            jax.ShapeDtypeStruct((_B, _D), jnp.float32),
            jax.ShapeDtypeStruct((_B, 2), jnp.float32),
        ),
        mesh=mesh,
        scratch_types=[
            pltpu.VMEM((bpw,), jnp.int32),
            pltpu.VMEM((bpw, _D), jnp.float32),
            pltpu.VMEM((bpw, 2), jnp.float32),
            pltpu.SemaphoreType.DMA,
            pltpu.SemaphoreType.DMA,
        ],
        compiler_params=pltpu.CompilerParams(needs_layout_passes=False),
    )
    def gather_kernel(node_hbm, psi_hbm, coords_hbm, psi_out, cc_out,
                      idx_v, rows_v, cc_v, sem_a, sem_b):
        wid = lax.axis_index("s") * nc + lax.axis_index("c")
        base = wid * bpw
        pltpu.sync_copy(node_hbm.at[pl.ds(base, bpw)], idx_v)
        for j in range(bpw // nl):
            node = idx_v[pl.ds(j * nl, nl)]
            row = base + j * nl + lax.iota(jnp.int32, nl)
            idx_v[pl.ds(j * nl, nl)] = row * _N + node
        cp_a = pltpu.async_copy(psi_hbm.at[idx_v], rows_v, sem_a)
        lanes = lax.iota(jnp.int32, nl)
        cps = []
        for j in range(bpw // nl):
            qv = idx_v[pl.ds(j * nl, nl)]
            for l in range(nl):
                q_i = jnp.sum(jnp.where(lanes == l, qv, 0))
                i = j * nl + l
                cps.append(pltpu.async_copy(
                    coords_hbm.at[pl.ds(q_i, 1), :],
                    cc_v.at[pl.ds(i, 1), :], sem_b))
        cp_a.wait()
        for cp in cps:
            cp.wait()
        pltpu.sync_copy(rows_v, psi_out.at[pl.ds(base, bpw)])
        pltpu.sync_copy(cc_v, cc_out.at[pl.ds(base, bpw)])

    return gather_kernel(current_node, psi_flat, coords_flat)


def _tc_project_body(psi_ref, cc_ref, cap_ref, used_ref, node_ref, tf_ref,
                     wq1_ref, wq2_ref, q_ref):
    live = (node_ref[...] != 0).astype(jnp.float32)
    psi = psi_ref[...] * live
    q = lax.dot_general(psi, wq1_ref[...],
                        (((1,), (1,)), ((), ())),
                        preferred_element_type=jnp.float32)
    cap = cap_ref[...]
    cap_norm = (cap - used_ref[...]) / jnp.maximum(cap, 1e-8)
    t_col = jnp.full((_B, 1), tf_ref[0, 0], jnp.float32)
    extras = jnp.concatenate([cap_norm, t_col, cc_ref[...]], axis=1)
    q = q + lax.dot_general(extras, wq2_ref[...],
                            (((1,), (1,)), ((), ())),
                            preferred_element_type=jnp.float32)
    q_ref[...] = q


def _tc_project(psi_curr, cc, cap, used, node, t_frac, wq1, wq2):
    return pl.pallas_call(
        _tc_project_body,
        out_shape=jax.ShapeDtypeStruct((_B, _D), jnp.float32),
        in_specs=[
            pl.BlockSpec(memory_space=pltpu.VMEM),
            pl.BlockSpec(memory_space=pltpu.VMEM),
            pl.BlockSpec(memory_space=pltpu.VMEM),
            pl.BlockSpec(memory_space=pltpu.VMEM),
            pl.BlockSpec(memory_space=pltpu.VMEM),
            pl.BlockSpec(memory_space=pltpu.SMEM),
            pl.BlockSpec(memory_space=pltpu.VMEM),
            pl.BlockSpec(memory_space=pltpu.VMEM),
        ],
        out_specs=pl.BlockSpec(memory_space=pltpu.VMEM),
    )(psi_curr, cc, cap, used, node, t_frac, wq1, wq2)


def kernel(psi_prime, current_node, capacity, used_capacity, coords, step,
           n_customers, Wq):
    psi_flat = psi_prime.reshape(_B * _N, _D)
    coords_flat = coords.reshape(_B * _N, 2)
    psi_curr, current_coords = _sc_gather(current_node, psi_flat, coords_flat)

    t_frac = (jnp.asarray(step, jnp.float32)
              / jnp.maximum(jnp.asarray(n_customers, jnp.float32), 1.0))
    t_frac = t_frac.reshape(1, 1)
    query = _tc_project(
        psi_curr,
        current_coords,
        capacity.reshape(_B, 1),
        used_capacity.reshape(_B, 1),
        current_node.reshape(_B, 1),
        t_frac,
        Wq[:, :_D],
        Wq[:, _D:],
    )
    return (query, current_coords)


# speedup vs baseline: 3.7242x; 3.7242x over previous
import functools

import jax
import jax.numpy as jnp
from jax import lax
from jax.experimental import pallas as pl
from jax.experimental.pallas import tpu as pltpu
from jax.experimental.pallas import tpu_sc as plsc

_B, _N, _D = 1024, 1000, 128


def _sc_gather(current_node, psi_flat, coords_flat):
    info = plsc.get_sparse_core_info()
    nc, ns, nl = info.num_cores, info.num_subcores, info.num_lanes
    nw = nc * ns
    bpw = _B // nw

    mesh = plsc.VectorSubcoreMesh(core_axis_name="c", subcore_axis_name="s")

    @functools.partial(
        pl.kernel,
        out_type=(
            jax.ShapeDtypeStruct((_B, _D), jnp.float32),
            jax.ShapeDtypeStruct((_B, 2), jnp.float32),
        ),
        mesh=mesh,
        scratch_types=[
            pltpu.VMEM((bpw,), jnp.int32),
            pltpu.VMEM((bpw, _D), jnp.float32),
            pltpu.VMEM((bpw, 2), jnp.float32),
            pltpu.SemaphoreType.DMA,
            pltpu.SemaphoreType.DMA,
        ],
        compiler_params=pltpu.CompilerParams(needs_layout_passes=False),
    )
    def gather_kernel(node_hbm, psi_hbm, coords_hbm, psi_out, cc_out,
                      idx_v, rows_v, cc_v, sem_a, sem_b):
        wid = lax.axis_index("s") * nc + lax.axis_index("c")
        base = wid * bpw
        pltpu.sync_copy(node_hbm.at[pl.ds(base, bpw)], idx_v)
        for j in range(bpw // nl):
            node = idx_v[pl.ds(j * nl, nl)]
            row = base + j * nl + lax.iota(jnp.int32, nl)
            idx_v[pl.ds(j * nl, nl)] = row * _N + node
        cp_a = pltpu.async_copy(psi_hbm.at[idx_v], rows_v, sem_a)
        lanes = lax.iota(jnp.int32, nl)
        cps = []
        for j in range(bpw // nl):
            qv = idx_v[pl.ds(j * nl, nl)]
            for l in range(nl):
                q_i = jnp.sum(jnp.where(lanes == l, qv, 0))
                i = j * nl + l
                n_i = q_i - (base + i) * _N
                cps.append(pltpu.async_copy(
                    coords_hbm.at[base + i, pl.ds(n_i, 1), :],
                    cc_v.at[pl.ds(i, 1), :], sem_b))
        cp_a.wait()
        for cp in cps:
            cp.wait()
        pltpu.sync_copy(rows_v, psi_out.at[pl.ds(base, bpw)])
        pltpu.sync_copy(cc_v, cc_out.at[pl.ds(base, bpw)])

    return gather_kernel(current_node, psi_flat, coords_flat)


def _tc_project_body(psi_ref, cc_ref, cap_ref, used_ref, node_ref, tf_ref,
                     wq1_ref, wq2_ref, q_ref):
    live = (node_ref[...] != 0).astype(jnp.float32)
    psi = psi_ref[...] * live
    q = lax.dot_general(psi, wq1_ref[...],
                        (((1,), (1,)), ((), ())),
                        preferred_element_type=jnp.float32)
    cap = cap_ref[...]
    cap_norm = (cap - used_ref[...]) / jnp.maximum(cap, 1e-8)
    t_col = jnp.full((_B, 1), tf_ref[0, 0], jnp.float32)
    extras = jnp.concatenate([cap_norm, t_col, cc_ref[...]], axis=1)
    q = q + lax.dot_general(extras, wq2_ref[...],
                            (((1,), (1,)), ((), ())),
                            preferred_element_type=jnp.float32)
    q_ref[...] = q


def _tc_project(psi_curr, cc, cap, used, node, t_frac, wq1, wq2):
    return pl.pallas_call(
        _tc_project_body,
        out_shape=jax.ShapeDtypeStruct((_B, _D), jnp.float32),
        in_specs=[
            pl.BlockSpec(memory_space=pltpu.VMEM),
            pl.BlockSpec(memory_space=pltpu.VMEM),
            pl.BlockSpec(memory_space=pltpu.VMEM),
            pl.BlockSpec(memory_space=pltpu.VMEM),
            pl.BlockSpec(memory_space=pltpu.VMEM),
            pl.BlockSpec(memory_space=pltpu.SMEM),
            pl.BlockSpec(memory_space=pltpu.VMEM),
            pl.BlockSpec(memory_space=pltpu.VMEM),
        ],
        out_specs=pl.BlockSpec(memory_space=pltpu.VMEM),
    )(psi_curr, cc, cap, used, node, t_frac, wq1, wq2)


def kernel(psi_prime, current_node, capacity, used_capacity, coords, step,
           n_customers, Wq):
    psi_flat = psi_prime.reshape(_B * _N, _D)
    psi_curr, current_coords = _sc_gather(current_node, psi_flat, coords)

    t_frac = (jnp.asarray(step, jnp.float32)
              / jnp.maximum(jnp.asarray(n_customers, jnp.float32), 1.0))
    t_frac = t_frac.reshape(1, 1)
    query = _tc_project(
        psi_curr,
        current_coords,
        capacity.reshape(_B, 1),
        used_capacity.reshape(_B, 1),
        current_node.reshape(_B, 1),
        t_frac,
        Wq[:, :_D],
        Wq[:, _D:],
    )
    return (query, current_coords)


# speedup vs baseline: 37.3652x; 10.0330x over previous
import functools

import jax
import jax.numpy as jnp
from jax import lax
from jax.experimental import pallas as pl
from jax.experimental.pallas import tpu as pltpu
from jax.experimental.pallas import tpu_sc as plsc

_B, _N, _D = 1024, 1000, 128


def _sc_gather(current_node, psi_flat, coords_flat):
    info = plsc.get_sparse_core_info()
    nc, ns, nl = info.num_cores, info.num_subcores, info.num_lanes
    nw = nc * ns
    bpw = _B // nw

    mesh = plsc.VectorSubcoreMesh(core_axis_name="c", subcore_axis_name="s")

    @functools.partial(
        pl.kernel,
        out_type=(
            jax.ShapeDtypeStruct((_B, _D), jnp.float32),
            jax.ShapeDtypeStruct((_B, 2), jnp.float32),
        ),
        mesh=mesh,
        scratch_types=[
            pltpu.VMEM((bpw,), jnp.int32),
            pltpu.VMEM((2 * bpw,), jnp.int32),
            pltpu.VMEM((bpw, _D), jnp.float32),
            pltpu.VMEM((2 * bpw, 128), jnp.float32),
            pltpu.VMEM((bpw, 2), jnp.float32),
            pltpu.SemaphoreType.DMA,
            pltpu.SemaphoreType.DMA,
        ],
        compiler_params=pltpu.CompilerParams(needs_layout_passes=False),
    )
    def gather_kernel(node_hbm, psi_hbm, zc_hbm, psi_out, cc_out,
                      idx_v, idx2_v, rows_v, cbuf_v, cc_v, sem_a, sem_b):
        wid = lax.axis_index("s") * nc + lax.axis_index("c")
        base = wid * bpw
        btile2 = lax.shift_right_logical(base, 7) * 2
        lane0 = base & 127
        pltpu.sync_copy(node_hbm.at[pl.ds(base, bpw)], idx_v)
        for j in range(bpw // nl):
            node = idx_v[pl.ds(j * nl, nl)]
            crow = node * 16 + btile2
            idx2_v[pl.ds(j * nl, nl)] = crow
            idx2_v[pl.ds(bpw + j * nl, nl)] = crow + 1
            row = base + j * nl + lax.iota(jnp.int32, nl)
            idx_v[pl.ds(j * nl, nl)] = row * _N + node
        cp_a = pltpu.async_copy(psi_hbm.at[idx_v], rows_v, sem_a)
        cp_b = pltpu.async_copy(zc_hbm.at[idx2_v], cbuf_v, sem_b)
        cp_a.wait()
        cp_b.wait()
        for c in range(2):
            for h in range(bpw // nl):
                iv = lax.iota(jnp.int32, nl)
                r = c * bpw + h * nl + iv
                col = lane0 + h * nl + iv
                vals = plsc.load_gather(cbuf_v, [r, col])
                plsc.store_scatter(
                    cc_v, [h * nl + iv, jnp.full((nl,), c, jnp.int32)], vals)
        pltpu.sync_copy(rows_v, psi_out.at[pl.ds(base, bpw)])
        pltpu.sync_copy(cc_v, cc_out.at[pl.ds(base, bpw)])

    return gather_kernel(current_node, psi_flat, coords_flat)


def _tc_project_body(psi_ref, cc_ref, cap_ref, used_ref, node_ref, tf_ref,
                     wq1_ref, wq2_ref, q_ref):
    live = (node_ref[...] != 0).astype(jnp.float32)
    psi = psi_ref[...] * live
    q = lax.dot_general(psi, wq1_ref[...],
                        (((1,), (1,)), ((), ())),
                        preferred_element_type=jnp.float32)
    cap = cap_ref[...]
    cap_norm = (cap - used_ref[...]) / jnp.maximum(cap, 1e-8)
    t_col = jnp.full((_B, 1), tf_ref[0, 0], jnp.float32)
    extras = jnp.concatenate([cap_norm, t_col, cc_ref[...]], axis=1)
    q = q + lax.dot_general(extras, wq2_ref[...],
                            (((1,), (1,)), ((), ())),
                            preferred_element_type=jnp.float32)
    q_ref[...] = q


def _tc_project(psi_curr, cc, cap, used, node, t_frac, wq1, wq2):
    return pl.pallas_call(
        _tc_project_body,
        out_shape=jax.ShapeDtypeStruct((_B, _D), jnp.float32),
        in_specs=[
            pl.BlockSpec(memory_space=pltpu.VMEM),
            pl.BlockSpec(memory_space=pltpu.VMEM),
            pl.BlockSpec(memory_space=pltpu.VMEM),
            pl.BlockSpec(memory_space=pltpu.VMEM),
            pl.BlockSpec(memory_space=pltpu.VMEM),
            pl.BlockSpec(memory_space=pltpu.SMEM),
            pl.BlockSpec(memory_space=pltpu.VMEM),
            pl.BlockSpec(memory_space=pltpu.VMEM),
        ],
        out_specs=pl.BlockSpec(memory_space=pltpu.VMEM),
    )(psi_curr, cc, cap, used, node, t_frac, wq1, wq2)


def kernel(psi_prime, current_node, capacity, used_capacity, coords, step,
           n_customers, Wq):
    psi_flat = psi_prime.reshape(_B * _N, _D)
    zc = (coords.transpose(1, 0, 2).reshape(_N, 8, 128, 2)
          .transpose(0, 1, 3, 2).reshape(_N * 16, 128))
    psi_curr, current_coords = _sc_gather(current_node, psi_flat, zc)

    t_frac = (jnp.asarray(step, jnp.float32)
              / jnp.maximum(jnp.asarray(n_customers, jnp.float32), 1.0))
    t_frac = t_frac.reshape(1, 1)
    query = _tc_project(
        psi_curr,
        current_coords,
        capacity.reshape(_B, 1),
        used_capacity.reshape(_B, 1),
        current_node.reshape(_B, 1),
        t_frac,
        Wq[:, :_D],
        Wq[:, _D:],
    )
    return (query, current_coords)
